# baseline (device time: 42628 ns/iter reference)
import jax
import jax.numpy as jnp
from jax import lax
from jax.experimental import pallas as pl
from jax.experimental.pallas import tpu as pltpu

N_DEV = 4
B, Sq, Skv, Hq, Dh = 2, 512, 512, 32, 64
H_LOC = Hq // N_DEV
D_LOC = H_LOC * Dh
D_MODEL = 768
CHUNK = (B * Sq) // N_DEV
HALF = CHUNK // 2
BLK = 64
N_HOPS = 2 * (N_DEV - 1)


def kernel(x, Wq, K_ext, V_ext, Wo):
    x16 = x.astype(jnp.bfloat16)
    wq16 = (Wq * 0.125).astype(jnp.bfloat16)
    k16 = jnp.transpose(K_ext, (0, 2, 1, 3)).astype(jnp.bfloat16)
    v16 = jnp.transpose(V_ext, (0, 2, 1, 3)).astype(jnp.bfloat16)
    wo16 = Wo.astype(jnp.bfloat16)

    def body(x_ref, wq_ref, k_ref, v_ref, wo_ref, out_ref,
             acc_r, acc_l, comm_r, comm_l,
             send_r, recv_r, send_l, recv_l):
        my_i = lax.axis_index("i")
        left = lax.rem(my_i + N_DEV - 1, N_DEV)
        right = lax.rem(my_i + 1, N_DEV)

        barrier_sem = pltpu.get_barrier_semaphore()
        pl.semaphore_signal(barrier_sem, inc=1, device_id=(left,),
                            device_id_type=pl.DeviceIdType.MESH)
        pl.semaphore_signal(barrier_sem, inc=1, device_id=(right,),
                            device_id_type=pl.DeviceIdType.MESH)
        pl.semaphore_wait(barrier_sem, 2)

        wq_loc = wq_ref[:, pl.ds(my_i * D_LOC, D_LOC)]
        wo_loc = wo_ref[pl.ds(my_i * D_LOC, D_LOC), :]

        def attn_bands(c, b, q16, s0g):
            band_ctx = []
            for t in range(2):
                r0 = s0g + t * HALF
                kvl = r0 + HALF
                q_t = q16[t * HALF:(t + 1) * HALF, :]
                row_blk = (lax.broadcasted_iota(jnp.int32, (HALF, kvl), 0)
                           + r0) // BLK
                col_blk = lax.broadcasted_iota(
                    jnp.int32, (HALF, kvl), 1) // BLK
                mask = col_blk <= row_blk
                ctx_cols = []
                for h in range(H_LOC):
                    q_h = q_t[:, h * Dh:(h + 1) * Dh]
                    k_h = k_ref[b, h, :kvl, :]
                    v_h = v_ref[b, h, :kvl, :]
                    s = lax.dot_general(
                        q_h, k_h, (((1,), (1,)), ((), ())),
                        preferred_element_type=jnp.float32)
                    e = jnp.exp(jnp.where(mask, s, -1e9))
                    denom = jnp.sum(e, axis=-1, keepdims=True)
                    ctx_h = lax.dot_general(
                        e.astype(jnp.bfloat16), v_h,
                        (((1,), (0,)), ((), ())),
                        preferred_element_type=jnp.float32)
                    ctx_cols.append((ctx_h / denom).astype(jnp.bfloat16))
                band_ctx.append(jnp.concatenate(ctx_cols, axis=-1))
            ctx = jnp.concatenate(band_ctx, axis=0)
            part = jnp.dot(ctx, wo_loc,
                           preferred_element_type=jnp.float32)
            p16 = part.astype(jnp.bfloat16)
            acc_r[c] = p16[:HALF, :]
            acc_l[c] = p16[HALF:, :]

        def compute_chunk(c):
            b = lax.div(c, 2)
            par = lax.rem(c, 2)
            xb = x_ref[b, pl.ds(par * CHUNK, CHUNK), :]
            q = jnp.dot(xb, wq_loc,
                        preferred_element_type=jnp.float32)
            q16 = q.astype(jnp.bfloat16)

            @pl.when(par == 0)
            def _():
                attn_bands(c, b, q16, 0)

            @pl.when(par == 1)
            def _():
                attn_bands(c, b, q16, CHUNK)

        def out_store(c, val_r, val_l):
            b = lax.div(c, 2)
            s0 = lax.rem(c, 2) * CHUNK
            if val_r is not None:
                out_ref[b, pl.ds(s0, HALF), :] = val_r.astype(jnp.float32)
            if val_l is not None:
                out_ref[b, pl.ds(s0 + HALF, HALF), :] = val_l.astype(
                    jnp.float32)

        rdmas = []

        def start_hop(hop, src_r, src_l):
            r = pltpu.make_async_remote_copy(
                src_ref=src_r, dst_ref=comm_r.at[hop],
                send_sem=send_r.at[hop], recv_sem=recv_r.at[hop],
                device_id=(right,), device_id_type=pl.DeviceIdType.MESH)
            l = pltpu.make_async_remote_copy(
                src_ref=src_l, dst_ref=comm_l.at[hop],
                send_sem=send_l.at[hop], recv_sem=recv_l.at[hop],
                device_id=(left,), device_id_type=pl.DeviceIdType.MESH)
            r.start()
            l.start()
            rdmas.append((r, l))
            return r, l

        compute_chunk(my_i)
        r0, l0 = start_hop(0, acc_r.at[my_i], acc_l.at[my_i])
        compute_chunk(lax.rem(my_i + 1, N_DEV))
        compute_chunk(lax.rem(my_i + N_DEV - 1, N_DEV))
        r0.wait_recv()
        cr = lax.rem(my_i + N_DEV - 1, N_DEV)
        acc_r[cr] = acc_r[cr] + comm_r[0]
        l0.wait_recv()
        cl = lax.rem(my_i + 1, N_DEV)
        acc_l[cl] = acc_l[cl] + comm_l[0]

        r1, l1 = start_hop(1, acc_r.at[cr], acc_l.at[cl])
        compute_chunk(lax.rem(my_i + 2, N_DEV))
        r1.wait_recv()
        cr = lax.rem(my_i + N_DEV - 2, N_DEV)
        acc_r[cr] = acc_r[cr] + comm_r[1]
        l1.wait_recv()
        cl = lax.rem(my_i + 2, N_DEV)
        acc_l[cl] = acc_l[cl] + comm_l[1]

        r2, l2 = start_hop(2, acc_r.at[cr], acc_l.at[cl])
        r2.wait_recv()
        cr = lax.rem(my_i + N_DEV - 3, N_DEV)
        own_r = cr
        red_r = acc_r[cr] + comm_r[2]
        acc_r[cr] = red_r
        l2.wait_recv()
        cl = lax.rem(my_i + 3, N_DEV)
        own_l = cl
        red_l = acc_l[cl] + comm_l[2]
        acc_l[cl] = red_l

        a0r, a0l = start_hop(3, acc_r.at[own_r], acc_l.at[own_l])
        out_store(own_r, red_r, None)
        out_store(own_l, None, red_l)
        a0r.wait_recv()
        a0l.wait_recv()

        a1r, a1l = start_hop(4, comm_r.at[3], comm_l.at[3])
        out_store(lax.rem(my_i + 2 * N_DEV, N_DEV), comm_r[3], None)
        out_store(my_i, None, comm_l[3])
        a1r.wait_recv()
        a1l.wait_recv()

        a2r, a2l = start_hop(5, comm_r.at[4], comm_l.at[4])
        out_store(lax.rem(my_i + N_DEV - 1, N_DEV), comm_r[4], None)
        out_store(lax.rem(my_i + 1, N_DEV), None, comm_l[4])
        a2r.wait_recv()
        a2l.wait_recv()
        out_store(lax.rem(my_i + N_DEV - 2, N_DEV), comm_r[5], None)
        out_store(lax.rem(my_i + 2, N_DEV), None, comm_l[5])

        for r, l in rdmas:
            r.wait_send()
            l.wait_send()

    return pl.pallas_call(
        body,
        out_shape=jax.ShapeDtypeStruct((B, Sq, D_MODEL), jnp.float32),
        in_specs=[pl.BlockSpec(memory_space=pltpu.VMEM)] * 5,
        out_specs=pl.BlockSpec(memory_space=pltpu.VMEM),
        scratch_shapes=[
            pltpu.VMEM((N_DEV, HALF, D_MODEL), jnp.bfloat16),
            pltpu.VMEM((N_DEV, HALF, D_MODEL), jnp.bfloat16),
            pltpu.VMEM((N_HOPS, HALF, D_MODEL), jnp.bfloat16),
            pltpu.VMEM((N_HOPS, HALF, D_MODEL), jnp.bfloat16),
            pltpu.SemaphoreType.DMA((N_HOPS,)),
            pltpu.SemaphoreType.DMA((N_HOPS,)),
            pltpu.SemaphoreType.DMA((N_HOPS,)),
            pltpu.SemaphoreType.DMA((N_HOPS,)),
        ],
        compiler_params=pltpu.CompilerParams(collective_id=0),
    )(x16, wq16, k16, v16, wo16)
